# baseline (speedup 1.0000x reference)
import jax
import jax.numpy as jnp
from jax.experimental import pallas as pl
from jax.experimental.pallas import tpu as pltpu

_N = 20000
_NPAD = 20480
_ROWS = _NPAD // 128
_PRE = 2000
_POST = 100
_IOU_THR = 0.5
_SCORE_THR = 0.05
_IMIN = -2147483648


def _nms_kernel(x1_ref, y1_ref, x2_ref, y2_ref, sp_ref,
                ox1_ref, oy1_ref, ox2_ref, oy2_ref, osc_ref, oidx_ref):
    sp = sp_ref[...]
    mp = jnp.where(sp >= _SCORE_THR, sp, jnp.where(sp < -1.5, -2.0, -1.0))
    x1 = x1_ref[...]
    y1 = y1_ref[...]
    x2 = x2_ref[...]
    y2 = y2_ref[...]
    area = (x2 - x1) * (y2 - y1)
    flat = (jax.lax.broadcasted_iota(jnp.int32, (_ROWS, 128), 0) * 128
            + jax.lax.broadcasted_iota(jnp.int32, (_ROWS, 128), 1))
    lane = jax.lax.broadcasted_iota(jnp.int32, (1, 128), 1)

    bits = jax.lax.bitcast_convert_type(mp, jnp.int32)
    key = jnp.where(mp >= 0.0, bits,
                    jnp.where(mp > -1.5, _IMIN + 1, _IMIN))

    c_pos = jnp.sum((key >= 0).astype(jnp.int32))
    zero = c_pos * 0

    def tsearch(k, t):
        t2 = t + (1 << (30 - k))
        c = jnp.sum((key >= t2).astype(jnp.int32))
        return jnp.where(c >= _PRE, t2, t)

    t_pos = jax.lax.fori_loop(0, 31, tsearch, zero)
    t_fin = jnp.where(c_pos >= _PRE, t_pos, _IMIN + 1)

    c1 = jnp.sum((key > t_fin).astype(jnp.int32))
    eq = key == t_fin
    r = _PRE - c1

    def fsearch(k, f):
        f2 = f + (1 << (14 - k))
        c = jnp.sum((eq & (flat < f2)).astype(jnp.int32))
        return jnp.where(c <= r, f2, f)

    f_fin = jax.lax.fori_loop(0, 15, fsearch, zero)

    candidate = (key > t_fin) | (eq & (flat < f_fin))
    cs = jnp.where(candidate, mp, -2.0)
    sa0 = jnp.where(candidate & (mp > 0.0), mp, -2.0)
    kept0 = jnp.zeros((_ROWS, 128), jnp.float32)

    ox1_ref[...] = jnp.zeros((1, 128), jnp.float32)
    oy1_ref[...] = jnp.zeros((1, 128), jnp.float32)
    ox2_ref[...] = jnp.zeros((1, 128), jnp.float32)
    oy2_ref[...] = jnp.zeros((1, 128), jnp.float32)
    osc_ref[...] = jnp.full((1, 128), -1.0, jnp.float32)
    oidx_ref[...] = jnp.zeros((1, 128), jnp.int32)

    def pivot_of(vals, m):
        return jnp.min(jnp.where(vals == m, flat, _NPAD))

    def pivot_coords(p):
        rp = p // 128
        ohc = (lane == p % 128).astype(jnp.float32)
        x1i = jnp.sum(x1_ref[pl.ds(rp, 1), :] * ohc)
        y1i = jnp.sum(y1_ref[pl.ds(rp, 1), :] * ohc)
        x2i = jnp.sum(x2_ref[pl.ds(rp, 1), :] * ohc)
        y2i = jnp.sum(y2_ref[pl.ds(rp, 1), :] * ohc)
        return x1i, y1i, x2i, y2i

    def emit(cnt, p, coords, sci):
        x1i, y1i, x2i, y2i = coords
        oh = lane == cnt
        ox1_ref[...] = jnp.where(oh, x1i, ox1_ref[...])
        oy1_ref[...] = jnp.where(oh, y1i, oy1_ref[...])
        ox2_ref[...] = jnp.where(oh, x2i, ox2_ref[...])
        oy2_ref[...] = jnp.where(oh, y2i, oy2_ref[...])
        osc_ref[...] = jnp.where(oh, sci, osc_ref[...])
        oidx_ref[...] = jnp.where(oh, p, oidx_ref[...])

    def cond(state):
        cnt, m = state[0], state[1]
        return (cnt < _POST) & (m > 0.0)

    def body(state):
        cnt, m, sa, kept = state
        p = pivot_of(sa, m)
        ohp = flat == p
        coords = pivot_coords(p)
        emit(cnt, p, coords, m)
        x1i, y1i, x2i, y2i = coords
        area_i = (x2i - x1i) * (y2i - y1i)
        xx1 = jnp.maximum(x1i, x1)
        yy1 = jnp.maximum(y1i, y1)
        xx2 = jnp.minimum(x2i, x2)
        yy2 = jnp.minimum(y2i, y2)
        inter = (jnp.clip(xx2 - xx1, 0.0, None)
                 * jnp.clip(yy2 - yy1, 0.0, None))
        union = area_i + area - inter
        iou = inter / jnp.maximum(union, 1e-8)
        after = (cs < m) | ((cs == m) & (flat > p))
        kill = ((iou > _IOU_THR) & after) | ohp
        sa_new = jnp.where(kill, -2.0, sa)
        kept_new = jnp.where(ohp, 1.0, kept)
        return cnt + 1, jnp.max(sa_new), sa_new, kept_new

    m0 = jnp.max(sa0)
    cnt_fin, _, _, kept_fin = jax.lax.while_loop(
        cond, body, (zero, m0, sa0, kept0))

    pa0 = jnp.where(candidate & (kept_fin == 0.0), cs, -2.0)

    def pad_cond(state):
        cnt, m = state[0], state[1]
        return (cnt < _POST) & (m > -1.5)

    def pad_body(state):
        cnt, m, pa = state
        p = pivot_of(pa, m)
        ohp = flat == p
        emit(cnt, p, pivot_coords(p), -1.0)
        pa_new = jnp.where(ohp, -2.0, pa)
        return cnt + 1, jnp.max(pa_new), pa_new

    jax.lax.while_loop(pad_cond, pad_body,
                       (cnt_fin, jnp.max(pa0), pa0))


def kernel(boxes, scores):
    pad = _NPAD - _N
    sp = jnp.pad(scores, (0, pad), constant_values=-2.0).reshape(_ROWS, 128)
    bp = jnp.pad(boxes, ((0, pad), (0, 0)))
    x1 = bp[:, 0].reshape(_ROWS, 128)
    y1 = bp[:, 1].reshape(_ROWS, 128)
    x2 = bp[:, 2].reshape(_ROWS, 128)
    y2 = bp[:, 3].reshape(_ROWS, 128)

    out_shapes = [jax.ShapeDtypeStruct((1, 128), jnp.float32)] * 5 + [
        jax.ShapeDtypeStruct((1, 128), jnp.int32)
    ]
    ox1, oy1, ox2, oy2, osc, oidx = pl.pallas_call(
        _nms_kernel,
        out_shape=out_shapes,
    )(x1, y1, x2, y2, sp)

    sel_boxes = jnp.stack(
        [ox1[0, :_POST], oy1[0, :_POST], ox2[0, :_POST], oy2[0, :_POST]],
        axis=1,
    )
    return sel_boxes, osc[0, :_POST], oidx[0, :_POST]


# speedup vs baseline: 1.0244x; 1.0244x over previous
import jax
import jax.numpy as jnp
from jax.experimental import pallas as pl
from jax.experimental.pallas import tpu as pltpu

_N = 20000
_NPAD = 20480
_ROWS = _NPAD // 128
_PRE = 2000
_POST = 100
_IOU_THR = 0.5
_SCORE_THR = 0.05
_IMIN = -2147483648


def _nms_kernel(x1_ref, y1_ref, x2_ref, y2_ref, area_ref, mp_ref,
                ox1_ref, oy1_ref, ox2_ref, oy2_ref, osc_ref, oidx_ref):
    mp = mp_ref[...]
    x1 = x1_ref[...]
    y1 = y1_ref[...]
    x2 = x2_ref[...]
    y2 = y2_ref[...]
    area = area_ref[...]
    flat = (jax.lax.broadcasted_iota(jnp.int32, (_ROWS, 128), 0) * 128
            + jax.lax.broadcasted_iota(jnp.int32, (_ROWS, 128), 1))
    lane = jax.lax.broadcasted_iota(jnp.int32, (1, 128), 1)

    bits = jax.lax.bitcast_convert_type(mp, jnp.int32)
    key = jnp.where(mp >= 0.0, bits,
                    jnp.where(mp > -1.5, _IMIN + 1, _IMIN))

    c_pos = jnp.sum((key >= 0).astype(jnp.int32))
    zero = c_pos * 0

    def tsearch(k, t):
        t2 = t + (1 << (30 - k))
        c = jnp.sum((key >= t2).astype(jnp.int32))
        return jnp.where(c >= _PRE, t2, t)

    t_pos = jax.lax.fori_loop(0, 31, tsearch, zero)
    t_fin = jnp.where(c_pos >= _PRE, t_pos, _IMIN + 1)

    c1 = jnp.sum((key > t_fin).astype(jnp.int32))
    eq = key == t_fin
    r = _PRE - c1

    def fsearch(k, f):
        f2 = f + (1 << (14 - k))
        c = jnp.sum((eq & (flat < f2)).astype(jnp.int32))
        return jnp.where(c <= r, f2, f)

    f_fin = jax.lax.fori_loop(0, 15, fsearch, zero)

    candidate = (key > t_fin) | (eq & (flat < f_fin))
    cs = jnp.where(candidate, mp, -2.0)
    sa0 = jnp.where(candidate & (mp > 0.0), mp, -2.0)
    kept0 = jnp.zeros((_ROWS, 128), jnp.float32)

    ox1_ref[...] = jnp.zeros((1, 128), jnp.float32)
    oy1_ref[...] = jnp.zeros((1, 128), jnp.float32)
    ox2_ref[...] = jnp.zeros((1, 128), jnp.float32)
    oy2_ref[...] = jnp.zeros((1, 128), jnp.float32)
    osc_ref[...] = jnp.full((1, 128), -1.0, jnp.float32)
    oidx_ref[...] = jnp.zeros((1, 128), jnp.int32)

    def pivot_of(vals, m):
        return jnp.min(jnp.where(vals == m, flat, _NPAD))

    def pivot_coords(p):
        rp = p // 128
        ohc = (lane == p % 128).astype(jnp.float32)
        x1i = jnp.sum(x1_ref[pl.ds(rp, 1), :] * ohc)
        y1i = jnp.sum(y1_ref[pl.ds(rp, 1), :] * ohc)
        x2i = jnp.sum(x2_ref[pl.ds(rp, 1), :] * ohc)
        y2i = jnp.sum(y2_ref[pl.ds(rp, 1), :] * ohc)
        return x1i, y1i, x2i, y2i

    def emit(cnt, p, coords, sci):
        x1i, y1i, x2i, y2i = coords
        oh = lane == cnt
        ox1_ref[...] = jnp.where(oh, x1i, ox1_ref[...])
        oy1_ref[...] = jnp.where(oh, y1i, oy1_ref[...])
        ox2_ref[...] = jnp.where(oh, x2i, ox2_ref[...])
        oy2_ref[...] = jnp.where(oh, y2i, oy2_ref[...])
        osc_ref[...] = jnp.where(oh, sci, osc_ref[...])
        oidx_ref[...] = jnp.where(oh, p, oidx_ref[...])

    def cond(state):
        cnt, m = state[0], state[1]
        return (cnt < _POST) & (m > 0.0)

    def body(state):
        cnt, m, sa, kept = state
        p = pivot_of(sa, m)
        ohp = flat == p
        coords = pivot_coords(p)
        emit(cnt, p, coords, m)
        x1i, y1i, x2i, y2i = coords
        area_i = (x2i - x1i) * (y2i - y1i)
        xx1 = jnp.maximum(x1i, x1)
        yy1 = jnp.maximum(y1i, y1)
        xx2 = jnp.minimum(x2i, x2)
        yy2 = jnp.minimum(y2i, y2)
        inter = (jnp.clip(xx2 - xx1, 0.0, None)
                 * jnp.clip(yy2 - yy1, 0.0, None))
        union = area_i + area - inter
        iou = inter / jnp.maximum(union, 1e-8)
        after = (cs < m) | ((cs == m) & (flat > p))
        kill = ((iou > _IOU_THR) & after) | ohp
        sa_new = jnp.where(kill, -2.0, sa)
        kept_new = jnp.where(ohp, 1.0, kept)
        return cnt + 1, jnp.max(sa_new), sa_new, kept_new

    m0 = jnp.max(sa0)
    cnt_fin, _, _, kept_fin = jax.lax.while_loop(
        cond, body, (zero, m0, sa0, kept0))

    pa0 = jnp.where(candidate & (kept_fin == 0.0), cs, -2.0)

    def pad_cond(state):
        cnt, m = state[0], state[1]
        return (cnt < _POST) & (m > -1.5)

    def pad_body(state):
        cnt, m, pa = state
        p = pivot_of(pa, m)
        ohp = flat == p
        emit(cnt, p, pivot_coords(p), -1.0)
        pa_new = jnp.where(ohp, -2.0, pa)
        return cnt + 1, jnp.max(pa_new), pa_new

    jax.lax.while_loop(pad_cond, pad_body,
                       (cnt_fin, jnp.max(pa0), pa0))


def kernel(boxes, scores):
    masked = jnp.where(scores >= _SCORE_THR, scores, -1.0)
    pad = _NPAD - _N
    mp = jnp.pad(masked, (0, pad), constant_values=-2.0).reshape(_ROWS, 128)
    bp = jnp.pad(boxes, ((0, pad), (0, 0)))
    x1 = bp[:, 0].reshape(_ROWS, 128)
    y1 = bp[:, 1].reshape(_ROWS, 128)
    x2 = bp[:, 2].reshape(_ROWS, 128)
    y2 = bp[:, 3].reshape(_ROWS, 128)
    area = (x2 - x1) * (y2 - y1)

    out_shapes = [jax.ShapeDtypeStruct((1, 128), jnp.float32)] * 5 + [
        jax.ShapeDtypeStruct((1, 128), jnp.int32)
    ]
    ox1, oy1, ox2, oy2, osc, oidx = pl.pallas_call(
        _nms_kernel,
        out_shape=out_shapes,
    )(x1, y1, x2, y2, area, mp)

    sel_boxes = jnp.stack(
        [ox1[0, :_POST], oy1[0, :_POST], ox2[0, :_POST], oy2[0, :_POST]],
        axis=1,
    )
    return sel_boxes, osc[0, :_POST], oidx[0, :_POST]
